# baseline (speedup 1.0000x reference)
import functools
import math

import jax
import jax.numpy as jnp
from jax import lax
from jax.experimental import pallas as pl
from jax.experimental.pallas import tpu as pltpu
from jax.experimental.pallas import tpu_sc as plsc

D_MODEL = 64
NUM_CORES = 2
NUM_SUBCORES = 16
NUM_WORKERS = NUM_CORES * NUM_SUBCORES
GROUP = 128
NBUF = 8
PREFETCH = 4
LANES = 16


@functools.lru_cache(maxsize=None)
def _build(b_total: int):
    groups_total = b_total // GROUP
    groups_per_w = groups_total // NUM_WORKERS
    n_col = D_MODEL // LANES

    mesh = plsc.VectorSubcoreMesh(
        core_axis_name="c",
        subcore_axis_name="s",
        num_cores=NUM_CORES,
        num_subcores=NUM_SUBCORES,
    )

    @functools.partial(
        pl.kernel,
        out_type=jax.ShapeDtypeStruct((b_total, D_MODEL), jnp.float32),
        mesh=mesh,
        scratch_types=[
            pltpu.VMEM((groups_per_w, GROUP), jnp.int32),
            pltpu.VMEM((NBUF, GROUP, D_MODEL), jnp.float32),
            pltpu.SemaphoreType.DMA((NBUF,)),
            pltpu.SemaphoreType.DMA((NBUF,)),
        ],
        compiler_params=pltpu.CompilerParams(use_tc_tiling_on_sc=False),
    )
    def emb_kernel(x_hbm, table_hbm, out_hbm, idx_v, rows_v, sem_g, sem_w):
        wid = lax.axis_index("s") * NUM_CORES + lax.axis_index("c")
        gbase = wid * groups_per_w

        pltpu.sync_copy(x_hbm.at[pl.ds(gbase, groups_per_w)], idx_v)

        def fire_gather(g, buf):
            pltpu.async_copy(
                table_hbm.at[idx_v.at[g]], rows_v.at[buf], sem_g.at[buf]
            )

        def wait_gather(buf):
            pltpu.make_async_copy(
                table_hbm.at[idx_v.at[0]], rows_v.at[buf], sem_g.at[buf]
            ).wait()

        def fire_write(g, buf):
            pltpu.async_copy(
                rows_v.at[buf],
                out_hbm.at[pl.ds((gbase + g) * GROUP, GROUP)],
                sem_w.at[buf],
            )

        def wait_write(buf):
            pltpu.make_async_copy(
                rows_v.at[buf], out_hbm.at[pl.ds(0, GROUP)], sem_w.at[buf]
            ).wait()

        for b in range(PREFETCH):
            fire_gather(b, b)

        @pl.loop(0, groups_per_w // NBUF)
        def _outer(t):
            for b in range(NBUF):
                g = t * NBUF + b
                gg = g + PREFETCH
                bp = (b + PREFETCH) % NBUF

                @pl.when(gg < groups_per_w)
                def _():
                    @pl.when(gg >= NBUF)
                    def _():
                        wait_write(bp)

                    fire_gather(gg, bp)

                wait_gather(b)

                @pl.loop(0, GROUP // 8)
                def _scale(i):
                    for rr in range(8):
                        for c in range(n_col):
                            sl = (b, i * 8 + rr, pl.ds(c * LANES, LANES))
                            rows_v[sl] = rows_v[sl] * 8.0

                fire_write(g, b)

        for b in range(NBUF):
            wait_write(b)

    return emb_kernel


def kernel(x, table):
    b_total = x.shape[0] * x.shape[1]
    xf = x.astype(jnp.int32).reshape(b_total // GROUP, GROUP)
    out = _build(b_total)(xf, table)
    return out.reshape(x.shape[0], x.shape[1], D_MODEL)
